# Initial kernel scaffold; baseline (speedup 1.0000x reference)
import functools

import jax
import jax.numpy as jnp
from jax import lax
from jax.experimental import pallas as pl
from jax.experimental.pallas import tpu as pltpu
from jax.experimental.pallas import tpu_sc as plsc

NC = 2
NS = 16
CH = 128


def _init_body(x_ref, w0_ref, b0_ref, gm_ref, bt_ref, w1_ref, b1_ref,
               w2_ref, b2_ref, h_ref, g_ref):
    h = jnp.dot(x_ref[...], w0_ref[...], preferred_element_type=jnp.float32)
    h = jnp.maximum(h + b0_ref[...], 0.0)
    h = gm_ref[...] * h * (1.0 / jnp.sqrt(1.0 + 1e-5)) + bt_ref[...]
    h = jnp.dot(h, w1_ref[...], preferred_element_type=jnp.float32)
    h = jnp.maximum(h + b1_ref[...], 0.0)
    h = jnp.dot(h, w2_ref[...], preferred_element_type=jnp.float32) + b2_ref[...]
    h_ref[...] = h
    i = pl.program_id(0)

    @pl.when(i == 0)
    def _():
        g_ref[...] = jnp.sum(h, axis=0, keepdims=True)

    @pl.when(i > 0)
    def _():
        g_ref[...] += jnp.sum(h, axis=0, keepdims=True)


def _node_body(p_ref, h_ref, g_ref, w0a_ref, w0b_ref, w0c_ref, b0_ref,
               w1_ref, b1_ref, w2_ref, b2_ref, ho_ref, go_ref):
    agg = p_ref[0] + p_ref[1]
    gvec = jnp.dot(g_ref[...], w0c_ref[...], preferred_element_type=jnp.float32) + b0_ref[...]
    t = (jnp.dot(agg, w0a_ref[...], preferred_element_type=jnp.float32)
         + jnp.dot(h_ref[...], w0b_ref[...], preferred_element_type=jnp.float32)
         + gvec)
    t = jnp.maximum(t, 0.0)
    t = jnp.dot(t, w1_ref[...], preferred_element_type=jnp.float32)
    t = jnp.maximum(t + b1_ref[...], 0.0)
    t = jnp.dot(t, w2_ref[...], preferred_element_type=jnp.float32) + b2_ref[...]
    nrm = jnp.sqrt(jnp.sum(t * t, axis=1, keepdims=True))
    o = t / nrm
    ho_ref[...] = o
    i = pl.program_id(0)

    @pl.when(i == 0)
    def _():
        go_ref[...] = jnp.sum(o, axis=0, keepdims=True)

    @pl.when(i > 0)
    def _():
        go_ref[...] += jnp.sum(o, axis=0, keepdims=True)


def _seg_body(n_nodes, n_chunks, h_hbm, src_hbm, dst_hbm, z_hbm, out_hbm,
              agg, idx_s, idx_d, rows, sem):
    cid = lax.axis_index("c")
    sid = lax.axis_index("s")
    w = sid * NC + cid
    nw = NC * NS
    rs = n_nodes // NS
    pltpu.sync_copy(z_hbm, agg.at[pl.ds(sid * rs, rs)])
    plsc.subcore_barrier()

    my_chunks = (n_chunks // nw) + jnp.where(w < (n_chunks % nw), 1, 0)

    def body(i, carry):
        base = (w + i * nw) * CH
        pltpu.sync_copy(src_hbm.at[pl.ds(base, CH)], idx_s)
        pltpu.sync_copy(dst_hbm.at[pl.ds(base, CH)], idx_d)
        pltpu.async_copy(h_hbm.at[idx_s], rows, sem).wait()
        pltpu.sync_copy(rows, agg.at[idx_d], add=True)
        return carry

    lax.fori_loop(0, my_chunks, body, 0)
    plsc.subcore_barrier()
    pltpu.sync_copy(agg.at[pl.ds(sid * rs, rs)],
                    out_hbm.at[cid, pl.ds(sid * rs, rs)])


def _make_seg_call(n_nodes, n_edges, hdim):
    n_chunks = n_edges // CH
    mesh = plsc.VectorSubcoreMesh(core_axis_name="c", subcore_axis_name="s",
                                  num_cores=NC, num_subcores=NS)
    return pl.kernel(
        functools.partial(_seg_body, n_nodes, n_chunks),
        out_type=jax.ShapeDtypeStruct((NC, n_nodes, hdim), jnp.float32),
        mesh=mesh,
        scratch_types=[
            pltpu.VMEM_SHARED((n_nodes, hdim), jnp.float32),
            pltpu.VMEM((CH,), jnp.int32),
            pltpu.VMEM((CH,), jnp.int32),
            pltpu.VMEM((CH, hdim), jnp.float32),
            pltpu.SemaphoreType.DMA,
        ],
    )


def kernel(x, edge_index, init_W0, init_b0, bn_gamma, bn_beta, init_W1,
           init_b1, init_W2, init_b2, node_W0, node_b0, node_W1, node_b1,
           node_W2, node_b2):
    n, d = x.shape
    e = edge_index.shape[1]
    hdim = init_W2.shape[1]
    blocks = node_W0.shape[0]
    iters = 3
    rblk = 1000
    grid = (n // rblk,)

    src = edge_index[0]
    dst = edge_index[1]
    zeros = jnp.zeros((n // NS, hdim), dtype=jnp.float32)

    row2 = lambda v: v.reshape(1, -1)

    def full(shape):
        return pl.BlockSpec(shape, lambda i: (0,) * len(shape))

    rows_in = pl.BlockSpec((rblk, d), lambda i: (i, 0))
    rows_out = pl.BlockSpec((rblk, hdim), lambda i: (i, 0))

    h, g = pl.pallas_call(
        _init_body,
        grid=grid,
        in_specs=[
            rows_in,
            full((d, init_W0.shape[1])),
            full((1, init_W0.shape[1])),
            full((1, init_W0.shape[1])),
            full((1, init_W0.shape[1])),
            full(init_W1.shape),
            full((1, init_W1.shape[1])),
            full(init_W2.shape),
            full((1, hdim)),
        ],
        out_specs=[rows_out, pl.BlockSpec((1, hdim), lambda i: (0, 0))],
        out_shape=[
            jax.ShapeDtypeStruct((n, hdim), jnp.float32),
            jax.ShapeDtypeStruct((1, hdim), jnp.float32),
        ],
    )(x, init_W0, row2(init_b0), row2(bn_gamma), row2(bn_beta), init_W1,
      row2(init_b1), init_W2, row2(init_b2))

    seg_call = _make_seg_call(n, e, hdim)

    mid = node_W1.shape[1]
    node_call = pl.pallas_call(
        _node_body,
        grid=grid,
        in_specs=[
            pl.BlockSpec((NC, rblk, hdim), lambda i: (0, i, 0)),
            rows_out,
            full((1, hdim)),
            full((hdim, mid)),
            full((hdim, mid)),
            full((hdim, mid)),
            full((1, mid)),
            full((mid, mid)),
            full((1, mid)),
            full((mid, hdim)),
            full((1, hdim)),
        ],
        out_specs=[rows_out, pl.BlockSpec((1, hdim), lambda i: (0, 0))],
        out_shape=[
            jax.ShapeDtypeStruct((n, hdim), jnp.float32),
            jax.ShapeDtypeStruct((1, hdim), jnp.float32),
        ],
    )

    for b in range(blocks):
        w0a = node_W0[b, :hdim]
        w0b = node_W0[b, hdim:2 * hdim]
        w0c = node_W0[b, 2 * hdim:]
        b0 = row2(node_b0[b])
        w1 = node_W1[b]
        b1 = row2(node_b1[b])
        w2 = node_W2[b]
        b2 = row2(node_b2[b])
        for _ in range(iters):
            p = seg_call(h, src, dst, zeros)
            h, g = node_call(p, h, g, w0a, w0b, w0c, b0, w1, b1, w2, b2)
    return h


# speedup vs baseline: 5.3819x; 5.3819x over previous
import functools

import jax
import jax.numpy as jnp
from jax import lax
from jax.experimental import pallas as pl
from jax.experimental.pallas import tpu as pltpu
from jax.experimental.pallas import tpu_sc as plsc

NC = 2
NS = 16
CH = 128


def _init_body(x_ref, w0_ref, b0_ref, gm_ref, bt_ref, w1_ref, b1_ref,
               w2_ref, b2_ref, h_ref, g_ref):
    h = jnp.dot(x_ref[...], w0_ref[...], preferred_element_type=jnp.float32)
    h = jnp.maximum(h + b0_ref[...], 0.0)
    h = gm_ref[...] * h * (1.0 / jnp.sqrt(1.0 + 1e-5)) + bt_ref[...]
    h = jnp.dot(h, w1_ref[...], preferred_element_type=jnp.float32)
    h = jnp.maximum(h + b1_ref[...], 0.0)
    h = jnp.dot(h, w2_ref[...], preferred_element_type=jnp.float32) + b2_ref[...]
    h_ref[...] = h
    i = pl.program_id(0)

    @pl.when(i == 0)
    def _():
        g_ref[...] = jnp.sum(h, axis=0, keepdims=True)

    @pl.when(i > 0)
    def _():
        g_ref[...] += jnp.sum(h, axis=0, keepdims=True)


def _node_body(p_ref, h_ref, g_ref, w0a_ref, w0b_ref, w0c_ref, b0_ref,
               w1_ref, b1_ref, w2_ref, b2_ref, ho_ref, go_ref):
    agg = p_ref[0] + p_ref[1]
    gvec = jnp.dot(g_ref[...], w0c_ref[...], preferred_element_type=jnp.float32) + b0_ref[...]
    t = (jnp.dot(agg, w0a_ref[...], preferred_element_type=jnp.float32)
         + jnp.dot(h_ref[...], w0b_ref[...], preferred_element_type=jnp.float32)
         + gvec)
    t = jnp.maximum(t, 0.0)
    t = jnp.dot(t, w1_ref[...], preferred_element_type=jnp.float32)
    t = jnp.maximum(t + b1_ref[...], 0.0)
    t = jnp.dot(t, w2_ref[...], preferred_element_type=jnp.float32) + b2_ref[...]
    nrm = jnp.sqrt(jnp.sum(t * t, axis=1, keepdims=True))
    o = t / nrm
    ho_ref[...] = o
    i = pl.program_id(0)

    @pl.when(i == 0)
    def _():
        go_ref[...] = jnp.sum(o, axis=0, keepdims=True)

    @pl.when(i > 0)
    def _():
        go_ref[...] += jnp.sum(o, axis=0, keepdims=True)


def _seg_body(rs, n_chunks, h_hbm, src_hbm, dst_hbm, z_hbm, out_hbm,
              agg, idx_s, idx_d, rows, sem):
    cid = lax.axis_index("c")
    sid = lax.axis_index("s")
    w = sid * NC + cid
    nw = NC * NS
    pltpu.sync_copy(z_hbm, agg.at[pl.ds(sid * rs, rs)])
    plsc.subcore_barrier()

    my_chunks = (n_chunks // nw) + jnp.where(w < (n_chunks % nw), 1, 0)

    def body(i, carry):
        base = (w + i * nw) * CH
        pltpu.sync_copy(src_hbm.at[pl.ds(base, CH)], idx_s)
        pltpu.sync_copy(dst_hbm.at[pl.ds(base, CH)], idx_d)
        pltpu.async_copy(h_hbm.at[idx_s], rows, sem).wait()
        pltpu.sync_copy(rows, agg.at[idx_d], add=True)
        return carry

    lax.fori_loop(0, my_chunks, body, 0)
    plsc.subcore_barrier()
    pltpu.sync_copy(agg.at[pl.ds(sid * rs, rs)],
                    out_hbm.at[cid, pl.ds(sid * rs, rs)])


def _make_seg_call(n_nodes, n_edges, hdim):
    n_chunks = n_edges // CH
    rs = (-(-n_nodes // NS) + 7) // 8 * 8
    n_pad = rs * NS
    mesh = plsc.VectorSubcoreMesh(core_axis_name="c", subcore_axis_name="s",
                                  num_cores=NC, num_subcores=NS)
    return pl.kernel(
        functools.partial(_seg_body, rs, n_chunks),
        out_type=jax.ShapeDtypeStruct((NC, n_pad, hdim), jnp.float32),
        mesh=mesh,
        scratch_types=[
            pltpu.VMEM_SHARED((n_pad, hdim), jnp.float32),
            pltpu.VMEM((CH,), jnp.int32),
            pltpu.VMEM((CH,), jnp.int32),
            pltpu.VMEM((CH, hdim), jnp.float32),
            pltpu.SemaphoreType.DMA,
        ],
    )


def kernel(x, edge_index, init_W0, init_b0, bn_gamma, bn_beta, init_W1,
           init_b1, init_W2, init_b2, node_W0, node_b0, node_W1, node_b1,
           node_W2, node_b2):
    n, d = x.shape
    e = edge_index.shape[1]
    hdim = init_W2.shape[1]
    blocks = node_W0.shape[0]
    iters = 3
    rblk = 1000
    grid = (n // rblk,)

    src = edge_index[0]
    dst = edge_index[1]
    zeros = jnp.zeros(((-(-n // NS) + 7) // 8 * 8, hdim), dtype=jnp.float32)

    row2 = lambda v: v.reshape(1, -1)

    def full(shape):
        return pl.BlockSpec(shape, lambda i: (0,) * len(shape))

    rows_in = pl.BlockSpec((rblk, d), lambda i: (i, 0))
    rows_out = pl.BlockSpec((rblk, hdim), lambda i: (i, 0))

    h, g = pl.pallas_call(
        _init_body,
        grid=grid,
        in_specs=[
            rows_in,
            full((d, init_W0.shape[1])),
            full((1, init_W0.shape[1])),
            full((1, init_W0.shape[1])),
            full((1, init_W0.shape[1])),
            full(init_W1.shape),
            full((1, init_W1.shape[1])),
            full(init_W2.shape),
            full((1, hdim)),
        ],
        out_specs=[rows_out, pl.BlockSpec((1, hdim), lambda i: (0, 0))],
        out_shape=[
            jax.ShapeDtypeStruct((n, hdim), jnp.float32),
            jax.ShapeDtypeStruct((1, hdim), jnp.float32),
        ],
    )(x, init_W0, row2(init_b0), row2(bn_gamma), row2(bn_beta), init_W1,
      row2(init_b1), init_W2, row2(init_b2))

    seg_call = _make_seg_call(n, e, hdim)

    mid = node_W1.shape[1]
    node_call = pl.pallas_call(
        _node_body,
        grid=grid,
        in_specs=[
            pl.BlockSpec((NC, rblk, hdim), lambda i: (0, i, 0)),
            rows_out,
            full((1, hdim)),
            full((hdim, mid)),
            full((hdim, mid)),
            full((hdim, mid)),
            full((1, mid)),
            full((mid, mid)),
            full((1, mid)),
            full((mid, hdim)),
            full((1, hdim)),
        ],
        out_specs=[rows_out, pl.BlockSpec((1, hdim), lambda i: (0, 0))],
        out_shape=[
            jax.ShapeDtypeStruct((n, hdim), jnp.float32),
            jax.ShapeDtypeStruct((1, hdim), jnp.float32),
        ],
    )

    for b in range(blocks):
        w0a = node_W0[b, :hdim]
        w0b = node_W0[b, hdim:2 * hdim]
        w0c = node_W0[b, 2 * hdim:]
        b0 = row2(node_b0[b])
        w1 = node_W1[b]
        b1 = row2(node_b1[b])
        w2 = node_W2[b]
        b2 = row2(node_b2[b])
        for _ in range(iters):
            p = seg_call(h, src, dst, zeros)
            h, g = node_call(p, h, g, w0a, w0b, w0c, b0, w1, b1, w2, b2)
    return h
